# baseline (speedup 1.0000x reference)
import jax
import jax.numpy as jnp
from jax import lax
from jax.experimental import pallas as pl
from jax.experimental.pallas import tpu as pltpu
from jax.experimental.pallas import tpu_sc as plsc

NC, NS, LANES = 2, 16, 16
NW = NC * NS
NBUF = 2
WROW = 128
TR = 8


def kernel(input_seq, input_mask, table):
    L, B = input_seq.shape
    V, D = table.shape
    BPW = B // NW
    KD = D // LANES
    KB = BPW // LANES
    LT = L // TR

    def body(seq_hbm, mask_hbm, table_hbm, out_hbm,
             seq_v, tgt_v, gb0, gb1, cnt_v, shacc,
             gs0, gs1, ss0, ss1):
        gb = (gb0, gb1)
        gsem = (gs0, gs1)
        ssem = (ss0, ss1)

        sid = lax.axis_index("s")
        wid = sid * NC + lax.axis_index("c")
        srow = sid * (BPW + 1)

        pltpu.sync_copy(seq_hbm.at[:, wid], seq_v)
        pltpu.sync_copy(mask_hbm.at[:, wid], tgt_v)

        zero = jnp.zeros((LANES,), jnp.float32)

        def zbody(i, c):
            for k in range(WROW // LANES):
                gb0[i, pl.ds(k * LANES, LANES)] = zero
            return c

        lax.fori_loop(0, BPW, zbody, 0)
        pltpu.sync_copy(gb0, shacc.at[pl.ds(srow, BPW)])
        pltpu.sync_copy(gb0.at[pl.ds(0, 1)], shacc.at[pl.ds(srow + BPW, 1)])

        iotas = [jnp.arange(k * LANES, (k + 1) * LANES, dtype=jnp.int32)
                 for k in range(KB)]
        trash = jnp.full((LANES,), BPW, jnp.int32)

        def cbody(l, cnts):
            a = l // TR
            r = l - a * TR
            out = []
            for k in range(KB):
                m = tgt_v[a, r, pl.ds(k * LANES, LANES)]
                tgt_v[a, r, pl.ds(k * LANES, LANES)] = srow + jnp.where(
                    m != 0, iotas[k], trash)
                out.append(cnts[k] + m)
            return tuple(out)

        cnts = lax.fori_loop(
            0, L, cbody,
            tuple(jnp.zeros((LANES,), jnp.int32) for _ in range(KB)))
        for k in range(KB):
            cnt_v[pl.ds(k * LANES, LANES)] = 1.0 / cnts[k].astype(jnp.float32)

        def idx_row(l):
            a = l // TR
            return a, l - a * TR

        for b in range(NBUF):
            a, r = idx_row(b)
            pltpu.async_copy(table_hbm.at[seq_v.at[a, r]], gb[b], gsem[b])

        def step(l, b, issue_next):
            a = l // TR
            r = l - a * TR
            pltpu.make_async_copy(table_hbm.at[seq_v.at[a, r]], gb[b],
                                  gsem[b]).wait()
            pltpu.async_copy(gb[b], shacc.at[tgt_v.at[a, r]], ssem[b],
                             add=True)
            pltpu.make_async_copy(gb[b], shacc.at[tgt_v.at[a, r]],
                                  ssem[b]).wait()
            if issue_next:
                ln = l + NBUF
                an = ln // TR
                rn = ln - an * TR
                pltpu.async_copy(table_hbm.at[seq_v.at[an, rn]], gb[b],
                                 gsem[b])

        NG = L // NBUF

        def gbody(g, c):
            for b in range(NBUF):
                step(g * NBUF + b, b, True)
            return c

        lax.fori_loop(0, NG - 1, gbody, 0)
        for b in range(NBUF):
            step((NG - 1) * NBUF + b, b, False)

        pltpu.sync_copy(shacc.at[pl.ds(srow, BPW)], gb1)
        lane_iota = jnp.arange(LANES, dtype=jnp.int32)

        def dbody(i, c):
            grp = i // LANES
            lane = i - grp * LANES
            rv = cnt_v[pl.ds(grp * LANES, LANES)]
            w = jnp.sum(jnp.where(lane_iota == lane, rv, 0.0))
            wv = jnp.full((LANES,), w, jnp.float32)
            for k in range(KD):
                sl = pl.ds(k * LANES, LANES)
                gb1[i, sl] = gb1[i, sl] * wv
            return c

        lax.fori_loop(0, BPW, dbody, 0)

        for t in range(BPW // TR):
            pltpu.sync_copy(gb1.at[pl.ds(TR * t, TR)],
                            out_hbm.at[wid * (BPW // TR) + t])

    t128 = jnp.pad(table, ((0, 0), (0, WROW - D)))
    seq4 = jnp.transpose(
        jnp.reshape(input_seq, (LT, TR, B // WROW, WROW)), (0, 2, 1, 3))
    mask4 = jnp.transpose(
        jnp.reshape(input_mask, (LT, TR, B // WROW, WROW)), (0, 2, 1, 3))
    mesh = plsc.VectorSubcoreMesh(core_axis_name="c", subcore_axis_name="s",
                                  num_cores=NC, num_subcores=NS)
    run = pl.kernel(
        body,
        out_type=jax.ShapeDtypeStruct((B // TR, TR, WROW), jnp.float32),
        mesh=mesh,
        compiler_params=pltpu.CompilerParams(needs_layout_passes=False,
                                             use_tc_tiling_on_sc=True),
        scratch_types=[
            pltpu.VMEM((LT, TR, WROW), jnp.int32),
            pltpu.VMEM((LT, TR, WROW), jnp.int32),
            *[pltpu.VMEM((BPW, WROW), jnp.float32) for _ in range(NBUF)],
            pltpu.VMEM((BPW,), jnp.float32),
            pltpu.VMEM_SHARED((NS * (BPW + 1), WROW), jnp.float32),
            *[pltpu.SemaphoreType.DMA for _ in range(2 * NBUF)],
        ],
    )
    out = run(seq4, mask4, t128)
    return jnp.reshape(out, (B, WROW))[:, :D]
